# baseline (speedup 1.0000x reference)
import functools

import jax
import jax.numpy as jnp
import numpy as np
from jax.experimental import pallas as pl
from jax.experimental.pallas import tpu as pltpu
from jax.experimental.pallas import tpu_sc as plsc

N = 10000
D_IN = 128
E = 160000
M = 13
HID = 64
CD = 6
DIM = 64
G = 128
OUT = 10

NP = 10240
NC = 2
NS = 16
CH = 128
EPT = E // NS
NCH = 80
NCHH = 40
TPT = NP // NS
MI = 7

BN = 1024

OPC = 2
OPS = NCHH // OPC


def _u_body(h_ref, w_ref, a_ref, u_ref):
    p = jnp.dot(h_ref[...], w_ref[0], preferred_element_type=jnp.float32)
    q = jnp.tanh(jnp.sum(p * a_ref[0], axis=1, keepdims=True))
    u_ref[0] = p * q


def _u_table(h, w, a3):
    d = h.shape[1]
    return pl.pallas_call(
        _u_body,
        grid=(M, NP // BN),
        in_specs=[
            pl.BlockSpec((BN, d), lambda m, i: (i, 0)),
            pl.BlockSpec((1, d, HID), lambda m, i: (m, 0, 0)),
            pl.BlockSpec((1, 1, HID), lambda m, i: (m, 0, 0)),
        ],
        out_specs=pl.BlockSpec((1, BN, HID), lambda m, i: (m, i, 0)),
        out_shape=jax.ShapeDtypeStruct((M, NP, HID), jnp.float32),
    )(h, w, a3)


@functools.cache
def _get_sc_edge_kernel():
    mesh = plsc.VectorSubcoreMesh(core_axis_name="c", subcore_axis_name="s",
                                  num_cores=NC, num_subcores=NS)
    return functools.partial(
        pl.kernel,
        out_type=jax.ShapeDtypeStruct((M + 1, NP, HID), jnp.float32),
        mesh=mesh,
        scratch_types=[
            pltpu.VMEM((OPS, OPC * CH), jnp.int32),
            pltpu.VMEM((OPS, OPC * CH), jnp.int32),
            [pltpu.VMEM((OPC * CH, HID), jnp.float32) for _ in range(2)],
            pltpu.VMEM((HID, HID), jnp.float32),
            [pltpu.VMEM_SHARED((NP, HID), jnp.float32) for _ in range(2)],
            [pltpu.SemaphoreType.DMA for _ in range(2)],
            [pltpu.SemaphoreType.DMA for _ in range(2)],
        ],
        compiler_params=pltpu.CompilerParams(use_tc_tiling_on_sc=False),
    )(_sc_edge_body)


def _sc_edge_body(u_hbm, srcp_hbm, dstp_hbm, out_hbm,
                  sidx, didx, rows, zbuf, accs, gsem, dsem):
    c = jax.lax.axis_index("c")
    s = jax.lax.axis_index("s")
    row0 = s * TPT

    z16 = jnp.zeros((16,), jnp.float32)

    @pl.loop(0, HID)
    def _zinit(i):
        for j in range(HID // 16):
            zbuf[i, pl.ds(j * 16, 16)] = z16

    for r in range(TPT // HID):
        pltpu.sync_copy(zbuf, accs[0].at[pl.ds(row0 + r * HID, HID)])
    plsc.subcore_barrier()

    for k in range(MI):
        acc = accs[k % 2]
        nh = 2 if k < MI - 1 else 1

        for hf in range(nh):
            rb = ((c * MI + k) * NS + s) * 2 + hf
            pltpu.sync_copy(srcp_hbm.at[rb], sidx)
            pltpu.sync_copy(dstp_hbm.at[rb], didx)

            pltpu.async_copy(u_hbm.at[sidx.at[0]], rows[0], gsem[0])

            @pl.loop(0, OPS, step=2)
            def _chunk(t):
                for i in range(2):
                    tt = t + i
                    bn = (i + 1) % 2

                    @pl.when(tt + 1 < OPS)
                    def _():
                        pltpu.async_copy(
                            u_hbm.at[sidx.at[tt + 1]], rows[bn], gsem[bn])

                    pltpu.make_async_copy(
                        u_hbm.at[sidx.at[tt]], rows[i], gsem[i]).wait()
                    pltpu.sync_copy(rows[i], acc.at[didx.at[tt]], add=True)

        plsc.subcore_barrier()
        slot = c * (MI - 1) + k if k < MI - 1 else M - 1 + c
        pltpu.async_copy(acc.at[pl.ds(row0, TPT)],
                         out_hbm.at[slot, pl.ds(row0, TPT)], dsem[k % 2])

        if k + 1 < MI:
            nxt = accs[(k + 1) % 2]
            if k >= 1:
                pltpu.make_async_copy(
                    nxt.at[pl.ds(row0, TPT)],
                    out_hbm.at[0, pl.ds(row0, TPT)], dsem[(k + 1) % 2]).wait()
            for r in range(TPT // HID):
                pltpu.sync_copy(zbuf, nxt.at[pl.ds(row0 + r * HID, HID)])
            plsc.subcore_barrier()

    for k in (MI - 2, MI - 1):
        pltpu.make_async_copy(
            accs[k % 2].at[pl.ds(row0, TPT)],
            out_hbm.at[0, pl.ds(row0, TPT)], dsem[k % 2]).wait()


def _mix_body(agg_ref, cw1_ref, b1_ref, w2_ref, b2_ref, h_ref):
    acc = jnp.broadcast_to(b1_ref[0], (BN, DIM))
    for m in range(M - 1):
        t = jnp.tanh(agg_ref[m])
        acc = acc + jnp.dot(t, cw1_ref[m], preferred_element_type=jnp.float32)
    t12 = jnp.tanh(agg_ref[M - 1] + agg_ref[M])
    acc = acc + jnp.dot(t12, cw1_ref[M - 1], preferred_element_type=jnp.float32)
    hmid = jnp.maximum(acc, 0.0)
    hout = jnp.dot(hmid, w2_ref[...], preferred_element_type=jnp.float32) + b2_ref[0]
    h_ref[...] = jnp.maximum(hout, 0.0)


def _mix_mlp(agg, cw1, b1f, w2, b2):
    return pl.pallas_call(
        _mix_body,
        grid=(NP // BN,),
        in_specs=[
            pl.BlockSpec((M + 1, BN, HID), lambda i: (0, i, 0)),
            pl.BlockSpec((M, HID, DIM), lambda i: (0, 0, 0)),
            pl.BlockSpec((1, DIM), lambda i: (0, 0)),
            pl.BlockSpec((DIM, DIM), lambda i: (0, 0)),
            pl.BlockSpec((1, DIM), lambda i: (0, 0)),
        ],
        out_specs=pl.BlockSpec((BN, DIM), lambda i: (i, 0)),
        out_shape=jax.ShapeDtypeStruct((NP, DIM), jnp.float32),
    )(agg, cw1, b1f, w2, b2)


def _pool_head_body(h_ref, b_ref, l1_ref, b1_ref, l2_ref, b2_ref,
                    pooled_ref, out_ref):
    i = pl.program_id(0)

    @pl.when(i == 0)
    def _():
        pooled_ref[...] = jnp.zeros_like(pooled_ref)

    ids = b_ref[0, 0, :]
    gi = jax.lax.broadcasted_iota(jnp.int32, (G, BN), 0)
    oh = (gi == ids[None, :]).astype(jnp.float32)
    pooled_ref[...] += jnp.dot(oh, h_ref[...], preferred_element_type=jnp.float32)

    @pl.when(i == pl.num_programs(0) - 1)
    def _():
        pg = pooled_ref[...]
        hg = jnp.maximum(
            jnp.dot(pg, l1_ref[...], preferred_element_type=jnp.float32)
            + b1_ref[0], 0.0)
        logits = (jnp.dot(hg, l2_ref[...], preferred_element_type=jnp.float32)
                  + b2_ref[0])
        mx = jnp.max(logits, axis=1, keepdims=True)
        lse = jnp.log(jnp.sum(jnp.exp(logits - mx), axis=1, keepdims=True)) + mx
        out_ref[...] = logits - lse


def _pool_head(h, batch3, l1, b1, l2p, b2p):
    _, out = pl.pallas_call(
        _pool_head_body,
        grid=(NP // BN,),
        in_specs=[
            pl.BlockSpec((BN, DIM), lambda i: (i, 0)),
            pl.BlockSpec((1, 1, BN), lambda i: (i, 0, 0)),
            pl.BlockSpec((DIM, DIM), lambda i: (0, 0)),
            pl.BlockSpec((1, DIM), lambda i: (0, 0)),
            pl.BlockSpec((DIM, G), lambda i: (0, 0)),
            pl.BlockSpec((1, G), lambda i: (0, 0)),
        ],
        out_specs=[
            pl.BlockSpec((G, DIM), lambda i: (0, 0)),
            pl.BlockSpec((G, G), lambda i: (0, 0)),
        ],
        out_shape=[
            jax.ShapeDtypeStruct((G, DIM), jnp.float32),
            jax.ShapeDtypeStruct((G, G), jnp.float32),
        ],
    )(h, batch3, l1, b1, l2p, b2p)
    return out


def kernel(x, edge_indices, batch, params):
    p = params

    ei = edge_indices.astype(jnp.int32)
    src = ei[:, 0, :]
    dst = ei[:, 1, :]
    sf = src[:M - 1].reshape(NC, MI - 1, NS, EPT)
    df = dst[:M - 1].reshape(NC, MI - 1, NS, EPT)
    pad4 = ((0, 0), (0, 0), (0, 0), (0, NCH * CH - EPT))
    shift = (jnp.arange(M - 1, dtype=jnp.int32) * NP).reshape(NC, MI - 1, 1, 1)
    sf = jnp.pad(sf, pad4, constant_values=0) + shift
    df = jnp.pad(df, pad4, constant_values=NP - 1)
    pad3 = ((0, 0), (0, 0), (0, NCH * CH - E // NC // NS))
    s12 = jnp.pad(src[M - 1].reshape(NC, NS, E // NC // NS), pad3,
                  constant_values=0) + (M - 1) * NP
    d12 = jnp.pad(dst[M - 1].reshape(NC, NS, E // NC // NS), pad3,
                  constant_values=NP - 1)
    srcp = jnp.concatenate([sf, s12[:, None]], axis=1).reshape(
        NC * MI * NS * 2, OPS, OPC * CH)
    dstp = jnp.concatenate([df, d12[:, None]], axis=1).reshape(
        NC * MI * NS * 2, OPS, OPC * CH)

    hpad = jnp.pad(x, ((0, NP - N), (0, 0)))

    h = hpad
    for l in range(3):
        w = p['W%d' % l]
        a3 = p['a%d' % l].reshape(M, 1, HID)
        scale = p['bn%d_g' % l] * np.float32(1.0 / np.sqrt(1.0 + 1e-5))
        w1s = p['mlp%d_w1' % l].reshape(M, CD, DIM) * scale[None, None, :]
        cw1 = jnp.einsum('mhc,mcd->mhd', p['C%d' % l], w1s)
        b1f = (p['mlp%d_b1' % l] * scale + p['bn%d_b' % l]).reshape(1, DIM)
        w2 = p['mlp%d_w2' % l]
        b2 = p['mlp%d_b2' % l].reshape(1, DIM)

        u = _u_table(h, w, a3).reshape(M * NP, HID)
        agg = _get_sc_edge_kernel()(u, srcp, dstp)
        h = _mix_mlp(agg, cw1, b1f, w2, b2)

    batchp = jnp.pad(batch.astype(jnp.int32), (0, NP - N), constant_values=G)
    batch3 = batchp.reshape(NP // BN, 1, BN)
    l2p = jnp.pad(p['lin2_w'], ((0, 0), (0, G - OUT)))
    b2p = jnp.pad(p['lin2_b'], (0, G - OUT),
                  constant_values=-1e30).reshape(1, G)
    out = _pool_head(h, batch3, p['lin1_w'], p['lin1_b'].reshape(1, DIM),
                     l2p, b2p)
    return out[:, :OUT]


# speedup vs baseline: 1.0200x; 1.0200x over previous
import functools

import jax
import jax.numpy as jnp
import numpy as np
from jax.experimental import pallas as pl
from jax.experimental.pallas import tpu as pltpu
from jax.experimental.pallas import tpu_sc as plsc

N = 10000
D_IN = 128
E = 160000
M = 13
HID = 64
CD = 6
DIM = 64
G = 128
OUT = 10

NP = 10240
NC = 2
NS = 16
CH = 128
EPT = E // NS
NCH = 80
NCHH = 40
TPT = NP // NS
MI = 7

BN = 1024

RING = 4
HALF = RING // 2


def _u_body(h_ref, w_ref, a_ref, u_ref):
    p = jnp.dot(h_ref[...], w_ref[0], preferred_element_type=jnp.float32)
    q = jnp.tanh(jnp.sum(p * a_ref[0], axis=1, keepdims=True))
    u_ref[0] = p * q


def _u_table(h, w, a3):
    d = h.shape[1]
    return pl.pallas_call(
        _u_body,
        grid=(M, NP // BN),
        in_specs=[
            pl.BlockSpec((BN, d), lambda m, i: (i, 0)),
            pl.BlockSpec((1, d, HID), lambda m, i: (m, 0, 0)),
            pl.BlockSpec((1, 1, HID), lambda m, i: (m, 0, 0)),
        ],
        out_specs=pl.BlockSpec((1, BN, HID), lambda m, i: (m, i, 0)),
        out_shape=jax.ShapeDtypeStruct((M, NP, HID), jnp.float32),
    )(h, w, a3)


@functools.cache
def _get_sc_edge_kernel():
    mesh = plsc.VectorSubcoreMesh(core_axis_name="c", subcore_axis_name="s",
                                  num_cores=NC, num_subcores=NS)
    return functools.partial(
        pl.kernel,
        out_type=jax.ShapeDtypeStruct((M + 1, NP, HID), jnp.float32),
        mesh=mesh,
        scratch_types=[
            pltpu.VMEM((NCHH, CH), jnp.int32),
            pltpu.VMEM((NCHH, CH), jnp.int32),
            [pltpu.VMEM((CH, HID), jnp.float32) for _ in range(RING)],
            pltpu.VMEM((HID, HID), jnp.float32),
            [pltpu.VMEM_SHARED((NP, HID), jnp.float32) for _ in range(2)],
            [pltpu.SemaphoreType.DMA for _ in range(RING)],
            [pltpu.SemaphoreType.DMA for _ in range(2)],
        ],
        compiler_params=pltpu.CompilerParams(use_tc_tiling_on_sc=False),
    )(_sc_edge_body)


def _sc_edge_body(u_hbm, srcp_hbm, dstp_hbm, out_hbm,
                  sidx, didx, rows, zbuf, accs, gsem, dsem):
    c = jax.lax.axis_index("c")
    s = jax.lax.axis_index("s")
    row0 = s * TPT

    z16 = jnp.zeros((16,), jnp.float32)

    @pl.loop(0, HID)
    def _zinit(i):
        for j in range(HID // 16):
            zbuf[i, pl.ds(j * 16, 16)] = z16

    for r in range(TPT // HID):
        pltpu.sync_copy(zbuf, accs[0].at[pl.ds(row0 + r * HID, HID)])
    plsc.subcore_barrier()

    for k in range(MI):
        acc = accs[k % 2]
        nh = 2 if k < MI - 1 else 1

        for hf in range(nh):
            rb = ((c * MI + k) * NS + s) * 2 + hf
            pltpu.sync_copy(srcp_hbm.at[rb], sidx)
            pltpu.sync_copy(dstp_hbm.at[rb], didx)

            for b in range(HALF):
                pltpu.async_copy(u_hbm.at[sidx.at[b]], rows[b], gsem[b])

            @pl.loop(0, NCHH, step=RING)
            def _chunk(j):
                for i in range(RING):
                    jj = j + i
                    bn = (i + HALF) % RING

                    @pl.when(jj + HALF < NCHH)
                    def _():
                        pltpu.async_copy(
                            u_hbm.at[sidx.at[jj + HALF]], rows[bn], gsem[bn])

                    pltpu.make_async_copy(
                        u_hbm.at[sidx.at[jj]], rows[i], gsem[i]).wait()
                    pltpu.sync_copy(rows[i], acc.at[didx.at[jj]], add=True)

        plsc.subcore_barrier()
        slot = c * (MI - 1) + k if k < MI - 1 else M - 1 + c
        pltpu.async_copy(acc.at[pl.ds(row0, TPT)],
                         out_hbm.at[slot, pl.ds(row0, TPT)], dsem[k % 2])

        if k + 1 < MI:
            nxt = accs[(k + 1) % 2]
            if k >= 1:
                pltpu.make_async_copy(
                    nxt.at[pl.ds(row0, TPT)],
                    out_hbm.at[0, pl.ds(row0, TPT)], dsem[(k + 1) % 2]).wait()
            for r in range(TPT // HID):
                pltpu.sync_copy(zbuf, nxt.at[pl.ds(row0 + r * HID, HID)])
            plsc.subcore_barrier()

    for k in (MI - 2, MI - 1):
        pltpu.make_async_copy(
            accs[k % 2].at[pl.ds(row0, TPT)],
            out_hbm.at[0, pl.ds(row0, TPT)], dsem[k % 2]).wait()


def _mix_body(agg_ref, cw1_ref, b1_ref, w2_ref, b2_ref, h_ref):
    acc = jnp.broadcast_to(b1_ref[0], (BN, DIM))
    for m in range(M - 1):
        t = jnp.tanh(agg_ref[m])
        acc = acc + jnp.dot(t, cw1_ref[m], preferred_element_type=jnp.float32)
    t12 = jnp.tanh(agg_ref[M - 1] + agg_ref[M])
    acc = acc + jnp.dot(t12, cw1_ref[M - 1], preferred_element_type=jnp.float32)
    hmid = jnp.maximum(acc, 0.0)
    hout = jnp.dot(hmid, w2_ref[...], preferred_element_type=jnp.float32) + b2_ref[0]
    h_ref[...] = jnp.maximum(hout, 0.0)


def _mix_mlp(agg, cw1, b1f, w2, b2):
    return pl.pallas_call(
        _mix_body,
        grid=(NP // BN,),
        in_specs=[
            pl.BlockSpec((M + 1, BN, HID), lambda i: (0, i, 0)),
            pl.BlockSpec((M, HID, DIM), lambda i: (0, 0, 0)),
            pl.BlockSpec((1, DIM), lambda i: (0, 0)),
            pl.BlockSpec((DIM, DIM), lambda i: (0, 0)),
            pl.BlockSpec((1, DIM), lambda i: (0, 0)),
        ],
        out_specs=pl.BlockSpec((BN, DIM), lambda i: (i, 0)),
        out_shape=jax.ShapeDtypeStruct((NP, DIM), jnp.float32),
    )(agg, cw1, b1f, w2, b2)


def _pool_head_body(h_ref, b_ref, l1_ref, b1_ref, l2_ref, b2_ref,
                    pooled_ref, out_ref):
    i = pl.program_id(0)

    @pl.when(i == 0)
    def _():
        pooled_ref[...] = jnp.zeros_like(pooled_ref)

    ids = b_ref[0, 0, :]
    gi = jax.lax.broadcasted_iota(jnp.int32, (G, BN), 0)
    oh = (gi == ids[None, :]).astype(jnp.float32)
    pooled_ref[...] += jnp.dot(oh, h_ref[...], preferred_element_type=jnp.float32)

    @pl.when(i == pl.num_programs(0) - 1)
    def _():
        pg = pooled_ref[...]
        hg = jnp.maximum(
            jnp.dot(pg, l1_ref[...], preferred_element_type=jnp.float32)
            + b1_ref[0], 0.0)
        logits = (jnp.dot(hg, l2_ref[...], preferred_element_type=jnp.float32)
                  + b2_ref[0])
        mx = jnp.max(logits, axis=1, keepdims=True)
        lse = jnp.log(jnp.sum(jnp.exp(logits - mx), axis=1, keepdims=True)) + mx
        out_ref[...] = logits - lse


def _pool_head(h, batch3, l1, b1, l2p, b2p):
    _, out = pl.pallas_call(
        _pool_head_body,
        grid=(NP // BN,),
        in_specs=[
            pl.BlockSpec((BN, DIM), lambda i: (i, 0)),
            pl.BlockSpec((1, 1, BN), lambda i: (i, 0, 0)),
            pl.BlockSpec((DIM, DIM), lambda i: (0, 0)),
            pl.BlockSpec((1, DIM), lambda i: (0, 0)),
            pl.BlockSpec((DIM, G), lambda i: (0, 0)),
            pl.BlockSpec((1, G), lambda i: (0, 0)),
        ],
        out_specs=[
            pl.BlockSpec((G, DIM), lambda i: (0, 0)),
            pl.BlockSpec((G, G), lambda i: (0, 0)),
        ],
        out_shape=[
            jax.ShapeDtypeStruct((G, DIM), jnp.float32),
            jax.ShapeDtypeStruct((G, G), jnp.float32),
        ],
    )(h, batch3, l1, b1, l2p, b2p)
    return out


def kernel(x, edge_indices, batch, params):
    p = params

    ei = edge_indices.astype(jnp.int32)
    src = ei[:, 0, :]
    dst = ei[:, 1, :]
    sf = src[:M - 1].reshape(NC, MI - 1, NS, EPT)
    df = dst[:M - 1].reshape(NC, MI - 1, NS, EPT)
    pad4 = ((0, 0), (0, 0), (0, 0), (0, NCH * CH - EPT))
    shift = (jnp.arange(M - 1, dtype=jnp.int32) * NP).reshape(NC, MI - 1, 1, 1)
    sf = jnp.pad(sf, pad4, constant_values=0) + shift
    df = jnp.pad(df, pad4, constant_values=NP - 1)
    pad3 = ((0, 0), (0, 0), (0, NCH * CH - E // NC // NS))
    s12 = jnp.pad(src[M - 1].reshape(NC, NS, E // NC // NS), pad3,
                  constant_values=0) + (M - 1) * NP
    d12 = jnp.pad(dst[M - 1].reshape(NC, NS, E // NC // NS), pad3,
                  constant_values=NP - 1)
    srcp = jnp.concatenate([sf, s12[:, None]], axis=1).reshape(
        NC * MI * NS * 2, NCHH, CH)
    dstp = jnp.concatenate([df, d12[:, None]], axis=1).reshape(
        NC * MI * NS * 2, NCHH, CH)

    hpad = jnp.pad(x, ((0, NP - N), (0, 0)))

    h = hpad
    for l in range(3):
        w = p['W%d' % l]
        a3 = p['a%d' % l].reshape(M, 1, HID)
        scale = p['bn%d_g' % l] * np.float32(1.0 / np.sqrt(1.0 + 1e-5))
        w1s = p['mlp%d_w1' % l].reshape(M, CD, DIM) * scale[None, None, :]
        cw1 = jnp.einsum('mhc,mcd->mhd', p['C%d' % l], w1s)
        b1f = (p['mlp%d_b1' % l] * scale + p['bn%d_b' % l]).reshape(1, DIM)
        w2 = p['mlp%d_w2' % l]
        b2 = p['mlp%d_b2' % l].reshape(1, DIM)

        u = _u_table(h, w, a3).reshape(M * NP, HID)
        agg = _get_sc_edge_kernel()(u, srcp, dstp)
        h = _mix_mlp(agg, cw1, b1f, w2, b2)

    batchp = jnp.pad(batch.astype(jnp.int32), (0, NP - N), constant_values=G)
    batch3 = batchp.reshape(NP // BN, 1, BN)
    l2p = jnp.pad(p['lin2_w'], ((0, 0), (0, G - OUT)))
    b2p = jnp.pad(p['lin2_b'], (0, G - OUT),
                  constant_values=-1e30).reshape(1, G)
    out = _pool_head(h, batch3, p['lin1_w'], p['lin1_b'].reshape(1, DIM),
                     l2p, b2p)
    return out[:, :OUT]
